# baseline (device time: 38371 ns/iter reference)
import os

import jax
import jax.numpy as jnp
from jax import lax
from jax.experimental import pallas as pl
from jax.experimental.pallas import tpu as pltpu

N_CHUNKS = 12
try:
    with open(os.path.join(os.path.dirname(__file__), "kmode.txt")) as _f:
        _MODE = _f.read().strip() or "full"
except OSError:
    _MODE = "full"


def kernel(A, B):
    m, k = A.shape
    k2, n = B.shape
    assert k == k2
    assert m % N_CHUNKS == 0
    mc = m // N_CHUNKS
    scale = 6.0 * float(k) ** 0.5 / 127.0
    inv_scale = 1.0 / scale

    def body(
        a_ref, b_ref, out_hbm,
        out_vmem, send_buf, recv_buf,
        out_sems, send_sems, recv_sems,
    ):
        my_x = lax.axis_index("x")
        my_y = lax.axis_index("y")
        peer = (my_x, 1 - my_y)

        comm = _MODE in ("full", "commfirst", "sendonly")
        if comm:
            barrier_sem = pltpu.get_barrier_semaphore()
            pl.semaphore_signal(
                barrier_sem, inc=1, device_id=peer,
                device_id_type=pl.DeviceIdType.MESH,
            )
            pl.semaphore_wait(barrier_sem, 1)

        def make_rdma(c):
            rows = pl.ds(c * mc, mc)
            return pltpu.make_async_remote_copy(
                src_ref=send_buf.at[rows],
                dst_ref=recv_buf.at[rows],
                send_sem=send_sems.at[c],
                recv_sem=recv_sems.at[c],
                device_id=peer,
                device_id_type=pl.DeviceIdType.MESH,
            )

        rdmas = []
        if _MODE in ("commfirst", "sendonly"):
            for c in range(N_CHUNKS):
                rdma = make_rdma(c)
                rdma.start()
                rdmas.append(rdma)
        for c in range(N_CHUNKS):
            if _MODE == "sendonly":
                break
            rows = pl.ds(c * mc, mc)
            part = jnp.dot(
                a_ref[rows, :], b_ref[...],
                preferred_element_type=jnp.float32,
            )
            out_vmem[rows, :] = part
            q = jnp.clip(jnp.round(part * inv_scale), -127.0, 127.0)
            send_buf[rows, :] = q.astype(jnp.int8)
            if _MODE == "full":
                rdma = make_rdma(c)
                rdma.start()
                rdmas.append(rdma)

        out_copies = []
        for c in range(N_CHUNKS):
            rows = pl.ds(c * mc, mc)
            if comm:
                rdmas[c].wait_recv()
                out_vmem[rows, :] += (
                    recv_buf[rows, :].astype(jnp.float32) * scale
                )
            ocp = pltpu.make_async_copy(
                out_vmem.at[rows], out_hbm.at[rows], out_sems.at[c]
            )
            ocp.start()
            out_copies.append(ocp)
        if comm:
            for c in range(N_CHUNKS):
                rdmas[c].wait_send()
        for c in range(N_CHUNKS):
            out_copies[c].wait()

    return pl.pallas_call(
        body,
        out_shape=jax.ShapeDtypeStruct((m, n), jnp.float32),
        in_specs=[
            pl.BlockSpec(memory_space=pltpu.VMEM),
            pl.BlockSpec(memory_space=pltpu.VMEM),
        ],
        out_specs=pl.BlockSpec(memory_space=pltpu.MemorySpace.HBM),
        scratch_shapes=[
            pltpu.VMEM((m, n), jnp.float32),
            pltpu.VMEM((m, n), jnp.int8),
            pltpu.VMEM((m, n), jnp.int8),
            pltpu.SemaphoreType.DMA((N_CHUNKS,)),
            pltpu.SemaphoreType.DMA((N_CHUNKS,)),
            pltpu.SemaphoreType.DMA((N_CHUNKS,)),
        ],
        compiler_params=pltpu.CompilerParams(
            collective_id=0
            if _MODE in ("full", "commfirst", "sendonly")
            else None
        ),
    )(A, B)
